# baseline (speedup 1.0000x reference)
import functools

import jax
import jax.numpy as jnp
from jax import lax
from jax.experimental import pallas as pl
from jax.experimental.pallas import tpu as pltpu
from jax.experimental.pallas import tpu_sc as plsc

BATCH = 64
SEQ = 1025
PATCH = 1024
DIM = 768
KEEP = 512
OUT = KEEP + 1

NC, NS, L = 2, 16, 16
NW = NC * NS
RPT = BATCH // NW
TILE_ROWS = RPT * OUT
C = 57
NCHUNK = TILE_ROWS // C
assert NCHUNK * C == TILE_ROWS
NCHV = PATCH // L


def _body(
    x_hbm, noise_hbm, out_hbm, noise_v, idx_v, buf0, buf1, gsem0, gsem1, ssem0, ssem1
):
    wid = lax.axis_index("s") * NC + lax.axis_index("c")

    zeros = jnp.zeros((L,), jnp.int32)
    ones = jnp.full((L,), 1, jnp.int32)
    kvec = jnp.full((L,), KEEP, jnp.int32)
    lanes = lax.iota(jnp.int32, L)

    for r in range(RPT):
        b = wid * RPT + r
        pltpu.sync_copy(noise_hbm.at[b], noise_v)

        def count_le(t_vec):
            def cbody(i, acc):
                bits = plsc.bitcast(noise_v[pl.ds(i * L, L)], jnp.int32)
                return acc + plsc.all_reduce_population_count(bits <= t_vec)

            return lax.fori_loop(0, NCHV, cbody, zeros, unroll=4)

        def sbody(_, lohi):
            lo, hi = lohi
            mid = lo + lax.shift_right_logical(hi - lo, 1)
            pred = count_le(mid) >= kvec
            return jnp.where(pred, lo, mid + 1), jnp.where(pred, mid, hi)

        _, tstar = lax.fori_loop(
            0, 31, sbody, (zeros, jnp.full((L,), 0x7FFFFFFF, jnp.int32))
        )

        def cbody_lt(i, acc):
            bits = plsc.bitcast(noise_v[pl.ds(i * L, L)], jnp.int32)
            return acc + plsc.all_reduce_population_count(bits < tstar)

        m = lax.fori_loop(0, NCHV, cbody_lt, zeros, unroll=4)
        need_eq = kvec - m

        p0 = jnp.full((L,), r * OUT, jnp.int32)
        plsc.store_scatter(
            idx_v,
            [p0 // C, p0 % C],
            jnp.full((L,), b * SEQ, jnp.int32),
            mask=lanes == zeros,
        )

        def compact(i, carry):
            kept, eqs = carry
            bits = plsc.bitcast(noise_v[pl.ds(i * L, L)], jnp.int32)
            is_lt = bits < tstar
            is_eq = bits == tstar
            eq_i = jnp.where(is_eq, ones, zeros)
            eq_rank = plsc.cumsum(eq_i) - eq_i + eqs
            keep = is_lt | (is_eq & (eq_rank < need_eq))
            k_i = jnp.where(keep, ones, zeros)
            pos = plsc.cumsum(k_i) - k_i + kept
            p = pos + (r * OUT + 1)
            gidx = (i * L + b * SEQ) + lanes
            plsc.store_scatter(idx_v, [p // C, p % C], gidx, mask=keep)
            return (
                kept + plsc.all_reduce_population_count(keep),
                eqs + plsc.all_reduce_population_count(is_eq),
            )

        lax.fori_loop(0, NCHV, compact, (zeros, zeros))

    out_base = wid * TILE_ROWS

    def g_issue(c, buf, sem):
        pltpu.async_copy(x_hbm.at[idx_v.at[c]], buf, sem)

    def g_wait(c, buf, sem):
        pltpu.make_async_copy(x_hbm.at[idx_v.at[c]], buf, sem).wait()

    def s_issue(c, buf, sem):
        pltpu.async_copy(buf, out_hbm.at[pl.ds(out_base + c * C, C)], sem)

    def s_wait(c, buf, sem):
        pltpu.make_async_copy(
            buf, out_hbm.at[pl.ds(out_base + c * C, C)], sem
        ).wait()

    bufs = ((buf0, gsem0, ssem0), (buf1, gsem1, ssem1))
    g_issue(0, buf0, gsem0)

    def gbody(cc, _):
        for k in range(2):
            p_buf, p_gs, p_ss = bufs[k]
            q_buf, q_gs, q_ss = bufs[1 - k]
            c = cc * 2 + k
            g_wait(c, p_buf, p_gs)
            s_issue(c, p_buf, p_ss)

            @pl.when(c + 1 < NCHUNK)
            def _():
                @pl.when(c >= 1)
                def _():
                    s_wait(c - 1, q_buf, q_ss)

                g_issue(c + 1, q_buf, q_gs)

        return 0

    lax.fori_loop(0, NCHUNK // 2, gbody, 0)
    s_wait(NCHUNK - 2, buf0, ssem0)
    s_wait(NCHUNK - 1, buf1, ssem1)


@jax.jit
def _run(x_flat, noise):
    mesh = plsc.VectorSubcoreMesh(
        core_axis_name="c", subcore_axis_name="s", num_cores=NC, num_subcores=NS
    )
    f = pl.kernel(
        _body,
        out_type=jax.ShapeDtypeStruct((BATCH * OUT, DIM), jnp.float32),
        mesh=mesh,
        scratch_types=[
            pltpu.VMEM((PATCH,), jnp.float32),
            pltpu.VMEM((NCHUNK, C), jnp.int32),
            pltpu.VMEM((C, DIM), jnp.float32),
            pltpu.VMEM((C, DIM), jnp.float32),
            pltpu.SemaphoreType.DMA,
            pltpu.SemaphoreType.DMA,
            pltpu.SemaphoreType.DMA,
            pltpu.SemaphoreType.DMA,
        ],
        compiler_params=pltpu.CompilerParams(
            use_tc_tiling_on_sc=False, needs_layout_passes=False
        ),
    )
    return f(x_flat, noise)


def kernel(x, force_drop, noise):
    del force_drop
    out = _run(x.reshape(BATCH * SEQ, DIM), noise)
    return out.reshape(BATCH, OUT, DIM)


# speedup vs baseline: 2.1143x; 2.1143x over previous
import jax
import jax.numpy as jnp
from jax import lax
from jax.experimental import pallas as pl
from jax.experimental.pallas import tpu as pltpu
from jax.experimental.pallas import tpu_sc as plsc

BATCH = 64
SEQ = 1025
PATCH = 1024
DIM = 768
KEEP = 512
OUT = KEEP + 1

NC, NS, L = 2, 16, 16
NW = NC * NS
RPT = BATCH // NW
C = 64
NCHUNK = KEEP // C
NCHV = PATCH // L


def _body(x_hbm, noise_hbm, out_hbm, noise_v, idx0, idx1, buf0, buf1,
          gsem0, gsem1, ssem0, ssem1):
    wid = lax.axis_index("s") * NC + lax.axis_index("c")
    b0 = wid * RPT
    idxs = (idx0, idx1)

    zeros = jnp.zeros((L,), jnp.int32)
    ones = jnp.full((L,), 1, jnp.int32)
    kvec = jnp.full((L,), KEEP, jnp.int32)
    lanes = lax.iota(jnp.int32, L)

    for r in range(RPT):
        idx_v = idxs[r]
        pltpu.sync_copy(noise_hbm.at[pl.ds((b0 + r) * PATCH, PATCH)], noise_v)

        def count_le(t_vec):
            def cbody(i, acc):
                bits = plsc.bitcast(noise_v[pl.ds(i * L, L)], jnp.int32)
                return acc + plsc.all_reduce_population_count(bits <= t_vec)

            return lax.fori_loop(0, NCHV, cbody, zeros, unroll=4)

        def sbody(_, lohi):
            lo, hi = lohi
            mid = lo + lax.shift_right_logical(hi - lo, 1)
            pred = count_le(mid) >= kvec
            return jnp.where(pred, lo, mid + 1), jnp.where(pred, mid, hi)

        _, tstar = lax.fori_loop(
            0, 31, sbody, (zeros, jnp.full((L,), 0x7FFFFFFF, jnp.int32))
        )

        def cbody_lt(i, acc):
            bits = plsc.bitcast(noise_v[pl.ds(i * L, L)], jnp.int32)
            return acc + plsc.all_reduce_population_count(bits < tstar)

        m = lax.fori_loop(0, NCHV, cbody_lt, zeros, unroll=4)
        need_eq = kvec - m

        for j in range(4):
            off = jnp.full((L,), j * L, jnp.int32) + lanes
            plsc.store_scatter(
                idx_v, [jnp.full((L,), NCHUNK, jnp.int32), zeros, off], zeros
            )
        plsc.store_scatter(idx_v, [zeros, zeros, zeros], zeros,
                           mask=lanes == zeros)

        def compact(i, carry):
            kept, eqs = carry
            bits = plsc.bitcast(noise_v[pl.ds(i * L, L)], jnp.int32)
            is_lt = bits < tstar
            is_eq = bits == tstar
            eq_i = jnp.where(is_eq, ones, zeros)
            eq_rank = plsc.cumsum(eq_i) - eq_i + eqs
            keep = is_lt | (is_eq & (eq_rank < need_eq))
            k_i = jnp.where(keep, ones, zeros)
            pos = plsc.cumsum(k_i) - k_i + kept
            p = pos + 1
            lidx = i * L + lanes
            plsc.store_scatter(idx_v, [p // C, zeros, p % C], lidx, mask=keep)
            return (
                kept + plsc.all_reduce_population_count(keep),
                eqs + plsc.all_reduce_population_count(is_eq),
            )

        lax.fori_loop(0, NCHV, compact, (zeros, zeros))

    def g_issue(r, c, buf, sem):
        pltpu.async_copy(x_hbm.at[b0 + r].at[idxs[r].at[c, 0]], buf, sem)

    def g_wait(r, c, buf, sem):
        pltpu.make_async_copy(
            x_hbm.at[b0 + r].at[idxs[r].at[c, 0]], buf, sem
        ).wait()

    def s_issue(r, c, buf, sem):
        pltpu.async_copy(buf, out_hbm.at[b0 + r].at[pl.ds(c * C, C)], sem)

    def s_wait(r, c, buf, sem):
        pltpu.make_async_copy(
            buf, out_hbm.at[b0 + r].at[pl.ds(c * C, C)], sem
        ).wait()

    bufs = ((buf0, gsem0, ssem0), (buf1, gsem1, ssem1))
    NTOT = RPT * NCHUNK
    steps = [(t // NCHUNK, t % NCHUNK) for t in range(NTOT)]
    g_issue(*steps[0], buf0, gsem0)
    for t, (r, c) in enumerate(steps):
        p_buf, p_gs, p_ss = bufs[t % 2]
        q_buf, q_gs, q_ss = bufs[1 - t % 2]
        g_wait(r, c, p_buf, p_gs)
        s_issue(r, c, p_buf, p_ss)
        if t + 1 < NTOT:
            if t >= 1:
                s_wait(*steps[t - 1], q_buf, q_ss)
            g_issue(*steps[t + 1], q_buf, q_gs)
    s_wait(*steps[NTOT - 2], bufs[(NTOT - 2) % 2][0], bufs[(NTOT - 2) % 2][2])
    s_wait(*steps[NTOT - 1], bufs[(NTOT - 1) % 2][0], bufs[(NTOT - 1) % 2][2])

    for r in range(RPT):
        src = x_hbm.at[b0 + r].at[idxs[r].at[NCHUNK, 0, pl.ds(0, 8)]]
        dst = buf0.at[pl.ds(0, 8)]
        pltpu.async_copy(src, dst, gsem0)
        pltpu.make_async_copy(src, dst, gsem0).wait()
        pltpu.sync_copy(
            buf0.at[pl.ds(0, 1)], out_hbm.at[b0 + r].at[pl.ds(KEEP, 1)]
        )


@jax.jit
def _run(x, noise_flat):
    mesh = plsc.VectorSubcoreMesh(
        core_axis_name="c", subcore_axis_name="s", num_cores=NC, num_subcores=NS
    )
    f = pl.kernel(
        _body,
        out_type=jax.ShapeDtypeStruct((BATCH, OUT, DIM), jnp.float32),
        mesh=mesh,
        scratch_types=[
            pltpu.VMEM((PATCH,), jnp.float32),
            pltpu.VMEM((NCHUNK + 1, 1, C), jnp.int32),
            pltpu.VMEM((NCHUNK + 1, 1, C), jnp.int32),
            pltpu.VMEM((C, DIM), jnp.float32),
            pltpu.VMEM((C, DIM), jnp.float32),
            pltpu.SemaphoreType.DMA,
            pltpu.SemaphoreType.DMA,
            pltpu.SemaphoreType.DMA,
            pltpu.SemaphoreType.DMA,
        ],
        compiler_params=pltpu.CompilerParams(needs_layout_passes=False),
    )
    return f(x, noise_flat)


def kernel(x, force_drop, noise):
    del force_drop
    return _run(x, noise.reshape(BATCH * PATCH))


# speedup vs baseline: 5.6885x; 2.6905x over previous
import jax
import jax.numpy as jnp
from jax import lax
from jax.experimental import pallas as pl
from jax.experimental.pallas import tpu as pltpu
from jax.experimental.pallas import tpu_sc as plsc

BATCH = 64
SEQ = 1025
PATCH = 1024
DIM = 768
KEEP = 512
OUT = KEEP + 1

NC, NS, L = 2, 16, 16
NW = NC * NS
RPT = BATCH // NW
SPT = KEEP // NW
NCHV = PATCH // L
PMS = 1024


def _sel_body(noise_hbm, pm_hbm, noise_v, list_v):
    wid = lax.axis_index("s") * NC + lax.axis_index("c")
    b0 = wid * RPT

    zeros = jnp.zeros((L,), jnp.int32)
    ones = jnp.full((L,), 1, jnp.int32)
    kvec = jnp.full((L,), KEEP, jnp.int32)
    lanes = lax.iota(jnp.int32, L)

    for r in range(RPT):
        pltpu.sync_copy(noise_hbm.at[pl.ds((b0 + r) * PATCH, PATCH)], noise_v)

        def count_le(t_vec):
            def cbody(i, acc):
                bits = plsc.bitcast(noise_v[pl.ds(i * L, L)], jnp.int32)
                return acc + plsc.all_reduce_population_count(bits <= t_vec)

            return lax.fori_loop(0, NCHV, cbody, zeros, unroll=4)

        def sbody(_, lohi):
            lo, hi = lohi
            mid = lo + lax.shift_right_logical(hi - lo, 1)
            pred = count_le(mid) >= kvec
            return jnp.where(pred, lo, mid + 1), jnp.where(pred, mid, hi)

        _, tstar = lax.fori_loop(
            0, 31, sbody, (zeros, jnp.full((L,), 0x7FFFFFFF, jnp.int32))
        )

        def cbody_lt(i, acc):
            bits = plsc.bitcast(noise_v[pl.ds(i * L, L)], jnp.int32)
            return acc + plsc.all_reduce_population_count(bits < tstar)

        m = lax.fori_loop(0, NCHV, cbody_lt, zeros, unroll=4)
        need_eq = kvec - m

        plsc.store_scatter(list_v, [jnp.full((L,), OUT, jnp.int32) + lanes],
                           zeros, mask=lanes < jnp.full((L,), 7, jnp.int32))
        plsc.store_scatter(list_v, [zeros], zeros, mask=lanes == zeros)

        def compact(i, carry):
            kept, eqs = carry
            bits = plsc.bitcast(noise_v[pl.ds(i * L, L)], jnp.int32)
            is_lt = bits < tstar
            is_eq = bits == tstar
            eq_i = jnp.where(is_eq, ones, zeros)
            eq_rank = plsc.cumsum(eq_i) - eq_i + eqs
            keep = is_lt | (is_eq & (eq_rank < need_eq))
            k_i = jnp.where(keep, ones, zeros)
            pos = plsc.cumsum(k_i) - k_i + kept
            lidx = i * L + lanes
            plsc.store_scatter(list_v, [pos + 1], lidx, mask=keep)
            return (
                kept + plsc.all_reduce_population_count(keep),
                eqs + plsc.all_reduce_population_count(is_eq),
            )

        lax.fori_loop(0, NCHV, compact, (zeros, zeros))

        pltpu.sync_copy(list_v, pm_hbm.at[pl.ds((b0 + r) * PMS, OUT + 7)])


def _gat_body(x_hbm, pmT_hbm, out_hbm, blk_v, gidx, buf0, buf1,
              gsem0, gsem1, ssem0, ssem1):
    wid = lax.axis_index("s") * NC + lax.axis_index("c")
    t0 = wid * SPT
    lanes = lax.iota(jnp.int32, L)

    pltpu.sync_copy(pmT_hbm.at[pl.ds(t0, SPT)], blk_v)

    def build_gidx(tt):
        for j in range(BATCH // L):
            bvec = jnp.full((L,), j * L, jnp.int32) + lanes
            s = plsc.load_gather(blk_v, [jnp.full((L,), tt, jnp.int32), bvec])
            gidx[pl.ds(j * L, L)] = s * BATCH + bvec

    def g_issue(buf, sem):
        pltpu.async_copy(x_hbm.at[gidx], buf, sem)

    def g_wait(buf, sem):
        pltpu.make_async_copy(x_hbm.at[gidx], buf, sem).wait()

    def s_issue(t, buf, sem):
        pltpu.async_copy(buf, out_hbm.at[pl.ds(t * BATCH, BATCH)], sem)

    def s_wait(t, buf, sem):
        pltpu.make_async_copy(
            buf, out_hbm.at[pl.ds(t * BATCH, BATCH)], sem
        ).wait()

    bufs = ((buf0, gsem0, ssem0), (buf1, gsem1, ssem1))
    build_gidx(0)
    g_issue(buf0, gsem0)
    for tt in range(SPT):
        p_buf, p_gs, p_ss = bufs[tt % 2]
        q_buf, q_gs, q_ss = bufs[1 - tt % 2]
        g_wait(p_buf, p_gs)
        s_issue(t0 + tt, p_buf, p_ss)
        if tt + 1 < SPT:
            build_gidx(tt + 1)
            if tt >= 1:
                s_wait(t0 + tt - 1, q_buf, q_ss)
            g_issue(q_buf, q_gs)
    s_wait(t0 + SPT - 2, bufs[(SPT - 2) % 2][0], bufs[(SPT - 2) % 2][2])
    s_wait(t0 + SPT - 1, bufs[(SPT - 1) % 2][0], bufs[(SPT - 1) % 2][2])

    @pl.when(wid == 0)
    def _():
        pltpu.sync_copy(pmT_hbm.at[pl.ds(KEEP, SPT)], blk_v)
        build_gidx(0)
        g_issue(buf0, gsem0)
        g_wait(buf0, gsem0)
        s_issue(KEEP, buf0, ssem0)
        s_wait(KEEP, buf0, ssem0)


_mesh = plsc.VectorSubcoreMesh(
    core_axis_name="c", subcore_axis_name="s", num_cores=NC, num_subcores=NS
)


@jax.jit
def _run(x, noise):
    sel = pl.kernel(
        _sel_body,
        out_type=jax.ShapeDtypeStruct((BATCH * PMS,), jnp.int32),
        mesh=_mesh,
        scratch_types=[
            pltpu.VMEM((PATCH,), jnp.float32),
            pltpu.VMEM((OUT + 7,), jnp.int32),
        ],
        compiler_params=pltpu.CompilerParams(needs_layout_passes=False),
    )
    pm = sel(noise.reshape(BATCH * PATCH))
    pmT = pm.reshape(BATCH, PMS).T

    gat = pl.kernel(
        _gat_body,
        out_type=jax.ShapeDtypeStruct((OUT * BATCH, DIM), jnp.float32),
        mesh=_mesh,
        scratch_types=[
            pltpu.VMEM((L, BATCH), jnp.int32),
            pltpu.VMEM((BATCH,), jnp.int32),
            pltpu.VMEM((BATCH, DIM), jnp.float32),
            pltpu.VMEM((BATCH, DIM), jnp.float32),
            pltpu.SemaphoreType.DMA,
            pltpu.SemaphoreType.DMA,
            pltpu.SemaphoreType.DMA,
            pltpu.SemaphoreType.DMA,
        ],
        compiler_params=pltpu.CompilerParams(needs_layout_passes=False),
    )
    x2d = x.transpose(1, 0, 2).reshape(SEQ * BATCH, DIM)
    out2d = gat(x2d, pmT)
    return out2d.reshape(OUT, BATCH, DIM).transpose(1, 0, 2)


def kernel(x, force_drop, noise):
    del force_drop
    return _run(x, noise)


# speedup vs baseline: 5.8233x; 1.0237x over previous
import jax
import jax.numpy as jnp
from jax import lax
from jax.experimental import pallas as pl
from jax.experimental.pallas import tpu as pltpu
from jax.experimental.pallas import tpu_sc as plsc

BATCH = 64
SEQ = 1025
PATCH = 1024
DIM = 768
KEEP = 512
OUT = KEEP + 1

NC, NS, L = 2, 16, 16
NW = NC * NS
RPT = BATCH // NW
SPT = KEEP // NW
NCHV = PATCH // L
PMS = 1024


def _sel_body(noise_hbm, pm_hbm, noise_v, list_v):
    wid = lax.axis_index("s") * NC + lax.axis_index("c")
    b0 = wid * RPT

    zeros = jnp.zeros((L,), jnp.int32)
    ones = jnp.full((L,), 1, jnp.int32)
    kvec = jnp.full((L,), KEEP, jnp.int32)
    lanes = lax.iota(jnp.int32, L)

    for r in range(RPT):
        pltpu.sync_copy(noise_hbm.at[pl.ds((b0 + r) * PATCH, PATCH)], noise_v)

        def count_le(t_vec):
            def cbody(i, acc):
                bits = plsc.bitcast(noise_v[pl.ds(i * L, L)], jnp.int32)
                return acc + plsc.all_reduce_population_count(bits <= t_vec)

            return lax.fori_loop(0, NCHV, cbody, zeros, unroll=4)

        def sbody(_, lohi):
            lo, hi = lohi
            mid = lo + lax.shift_right_logical(hi - lo, 1)
            pred = count_le(mid) >= kvec
            return jnp.where(pred, lo, mid + 1), jnp.where(pred, mid, hi)

        _, tstar = lax.fori_loop(
            0, 31, sbody, (zeros, jnp.full((L,), 0x7FFFFFFF, jnp.int32))
        )

        def cbody_lt(i, acc):
            bits = plsc.bitcast(noise_v[pl.ds(i * L, L)], jnp.int32)
            return acc + plsc.all_reduce_population_count(bits < tstar)

        m = lax.fori_loop(0, NCHV, cbody_lt, zeros, unroll=4)
        need_eq = kvec - m

        plsc.store_scatter(list_v, [jnp.full((L,), OUT, jnp.int32) + lanes],
                           zeros, mask=lanes < jnp.full((L,), 7, jnp.int32))
        plsc.store_scatter(list_v, [zeros], zeros, mask=lanes == zeros)

        def compact(i, carry):
            kept, eqs = carry
            bits = plsc.bitcast(noise_v[pl.ds(i * L, L)], jnp.int32)
            is_lt = bits < tstar
            is_eq = bits == tstar
            eq_i = jnp.where(is_eq, ones, zeros)
            eq_rank = plsc.cumsum(eq_i) - eq_i + eqs
            keep = is_lt | (is_eq & (eq_rank < need_eq))
            k_i = jnp.where(keep, ones, zeros)
            pos = plsc.cumsum(k_i) - k_i + kept
            lidx = i * L + lanes
            plsc.store_scatter(list_v, [pos + 1], lidx, mask=keep)
            return (
                kept + plsc.all_reduce_population_count(keep),
                eqs + plsc.all_reduce_population_count(is_eq),
            )

        lax.fori_loop(0, NCHV, compact, (zeros, zeros))

        pltpu.sync_copy(list_v, pm_hbm.at[pl.ds((b0 + r) * PMS, OUT + 7)])


CH = 32
NSUB = SPT * (BATCH // CH)
NBUF = 4


def _gat_body(x_hbm, pmT_hbm, out_hbm, blk_v,
              g0, g1, g2, g3, b0, b1, b2, b3,
              gs0, gs1, gs2, gs3, ss0, ss1, ss2, ss3):
    wid = lax.axis_index("s") * NC + lax.axis_index("c")
    t0 = wid * SPT
    lanes = lax.iota(jnp.int32, L)
    gidxs = (g0, g1, g2, g3)
    bufs = (b0, b1, b2, b3)
    gsems = (gs0, gs1, gs2, gs3)
    ssems = (ss0, ss1, ss2, ss3)

    pltpu.sync_copy(pmT_hbm.at[pl.ds(t0, SPT)], blk_v)

    def build_gidx(i, n_rows=CH):
        tt, hb = i // 2, (i % 2) * CH
        gidx = gidxs[i % NBUF]
        for j in range(n_rows // L):
            bvec = jnp.full((L,), hb + j * L, jnp.int32) + lanes
            s = plsc.load_gather(blk_v, [jnp.full((L,), tt, jnp.int32), bvec])
            gidx[pl.ds(j * L, L)] = s * BATCH + bvec

    def orow(i):
        return (t0 + i // 2) * BATCH + (i % 2) * CH

    def g_issue(i):
        pltpu.async_copy(x_hbm.at[gidxs[i % NBUF]], bufs[i % NBUF],
                         gsems[i % NBUF])

    def g_wait(i):
        pltpu.make_async_copy(x_hbm.at[gidxs[i % NBUF]], bufs[i % NBUF],
                              gsems[i % NBUF]).wait()

    def s_issue(i):
        pltpu.async_copy(bufs[i % NBUF], out_hbm.at[pl.ds(orow(i), CH)],
                         ssems[i % NBUF])

    def s_wait(i):
        pltpu.make_async_copy(bufs[i % NBUF],
                              out_hbm.at[pl.ds(orow(i), CH)],
                              ssems[i % NBUF]).wait()

    for i in range(NSUB + 2):
        if i < NSUB:
            if i >= NBUF:
                s_wait(i - NBUF)
            build_gidx(i)
            g_issue(i)
        if i >= 2:
            g_wait(i - 2)
            s_issue(i - 2)
    for i in range(NSUB - NBUF, NSUB):
        s_wait(i)

    @pl.when(wid < 8)
    def _():
        pltpu.sync_copy(pmT_hbm.at[pl.ds(KEEP, 8)], blk_v.at[pl.ds(0, 8)])
        bvec = wid * 8 + lanes
        s = plsc.load_gather(blk_v, [jnp.zeros((L,), jnp.int32), bvec],
                             mask=lanes < jnp.full((L,), 8, jnp.int32))
        g0[pl.ds(0, L)] = s * BATCH + bvec
        src = x_hbm.at[g0.at[pl.ds(0, 8)]]
        pltpu.async_copy(src, b0.at[pl.ds(0, 8)], gs0)
        pltpu.make_async_copy(src, b0.at[pl.ds(0, 8)], gs0).wait()
        dst = out_hbm.at[pl.ds(KEEP * BATCH + wid * 8, 8)]
        pltpu.async_copy(b0.at[pl.ds(0, 8)], dst, ss0)
        pltpu.make_async_copy(b0.at[pl.ds(0, 8)], dst, ss0).wait()


_mesh = plsc.VectorSubcoreMesh(
    core_axis_name="c", subcore_axis_name="s", num_cores=NC, num_subcores=NS
)


@jax.jit
def _run(x, noise):
    sel = pl.kernel(
        _sel_body,
        out_type=jax.ShapeDtypeStruct((BATCH * PMS,), jnp.int32),
        mesh=_mesh,
        scratch_types=[
            pltpu.VMEM((PATCH,), jnp.float32),
            pltpu.VMEM((OUT + 7,), jnp.int32),
        ],
        compiler_params=pltpu.CompilerParams(needs_layout_passes=False),
    )
    pm = sel(noise.reshape(BATCH * PATCH))
    pmT = pm.reshape(BATCH, PMS).T

    gat = pl.kernel(
        _gat_body,
        out_type=jax.ShapeDtypeStruct((OUT * BATCH, DIM), jnp.float32),
        mesh=_mesh,
        scratch_types=(
            [pltpu.VMEM((L, BATCH), jnp.int32)]
            + [pltpu.VMEM((CH,), jnp.int32)] * NBUF
            + [pltpu.VMEM((CH, DIM), jnp.float32)] * NBUF
            + [pltpu.SemaphoreType.DMA] * (2 * NBUF)
        ),
        compiler_params=pltpu.CompilerParams(needs_layout_passes=False),
    )
    x2d = x.transpose(1, 0, 2).reshape(SEQ * BATCH, DIM)
    out2d = gat(x2d, pmT)
    return out2d.reshape(OUT, BATCH, DIM).transpose(1, 0, 2)


def kernel(x, force_drop, noise):
    del force_drop
    return _run(x, noise)
